# Initial kernel scaffold; baseline (speedup 1.0000x reference)
import jax
import jax.numpy as jnp
from jax.experimental import pallas as pl


def kernel(proposals, bounding_boxes, labels):
    raise NotImplementedError("write your pallas kernel here")


# speedup vs baseline: 1.5952x; 1.5952x over previous
import jax
import jax.numpy as jnp
from jax import lax
from jax.experimental import pallas as pl
from jax.experimental.pallas import tpu as pltpu

_N = 20064
_ROWS = 160
_LANES = 128
_NPAD = _ROWS * _LANES
_G = 64
_C = 21
_K = 64

_FG_THRESH = 0.7
_BG_HI = 0.5
_BG_LO = 0.1


def _proposal_target_kernel(gt_smem, planes_ref, gtall_ref,
                            rois_ref, labels_ref, bbox_ref,
                            asg_ref, fg_ref, bg_ref):
    f32 = jnp.float32
    i32 = jnp.int32

    row_i = lax.broadcasted_iota(i32, (8, _LANES), 0)
    lane_i = lax.broadcasted_iota(i32, (8, _LANES), 1)
    li = lax.broadcasted_iota(i32, (1, _LANES), 1)

    def iou_chunk(i, _):
        s = i * 8
        ax1 = planes_ref[0, pl.ds(s, 8), :]
        ay1 = planes_ref[1, pl.ds(s, 8), :]
        ax2 = planes_ref[2, pl.ds(s, 8), :]
        ay2 = planes_ref[3, pl.ds(s, 8), :]
        area_a = (ax2 - ax1 + 1.0) * (ay2 - ay1 + 1.0)
        maxv = jnp.full((8, _LANES), -1.0, f32)
        asg = jnp.zeros((8, _LANES), f32)

        def per_gt(g, carry):
            maxv, asg = carry
            bx1 = gt_smem[0, g]
            by1 = gt_smem[1, g]
            bx2 = gt_smem[2, g]
            by2 = gt_smem[3, g]
            area_b = (bx2 - bx1 + 1.0) * (by2 - by1 + 1.0)
            iw = jnp.maximum(
                jnp.minimum(ax2, bx2) - jnp.maximum(ax1, bx1) + 1.0, 0.0)
            ih = jnp.maximum(
                jnp.minimum(ay2, by2) - jnp.maximum(ay1, by1) + 1.0, 0.0)
            inter = iw * ih
            union = area_a + area_b - inter
            iou = inter / jnp.maximum(union, 1e-8)
            upd = iou > maxv
            asg = jnp.where(upd, g.astype(f32), asg)
            maxv = jnp.where(upd, iou, maxv)
            return maxv, asg

        maxv, asg = lax.fori_loop(0, _G, per_gt, (maxv, asg))

        flat = (s + row_i) * _LANES + lane_i
        valid = flat < _N
        fg = jnp.where(valid & (maxv >= _FG_THRESH), maxv, -1.0)
        fg = jnp.where(valid, fg, -2.0)
        bg = jnp.where(valid & (maxv < _BG_HI) & (maxv >= _BG_LO), maxv, -1.0)
        bg = jnp.where(valid, bg, -2.0)
        asg_ref[pl.ds(s, 8), :] = asg
        fg_ref[pl.ds(s, 8), :] = fg
        bg_ref[pl.ds(s, 8), :] = bg
        return 0

    lax.fori_loop(0, _ROWS // 8, iou_chunk, 0)

    frow_i = lax.broadcasted_iota(i32, (_ROWS, _LANES), 0)
    flane_i = lax.broadcasted_iota(i32, (_ROWS, _LANES), 1)
    flat_all = frow_i * _LANES + flane_i
    big = jnp.int32(1 << 30)

    def select_lowest_max(ref):
        v = ref[...]
        m = jnp.max(v)
        idx = jnp.min(jnp.where(v == m, flat_all, big))
        ref[...] = jnp.where(flat_all == idx, -2.0, v)
        return idx

    def extract_coords(r, lc):
        def ext(p):
            row = planes_ref[p, pl.ds(r, 1), :]
            return jnp.sum(jnp.where(lc, row, 0.0), axis=1, keepdims=True)
        return ext(0), ext(1), ext(2), ext(3)

    def step(j, _):
        idx = select_lowest_max(fg_ref)
        r = idx // _LANES
        c = idx % _LANES
        lc = li == c
        ex1, ey1, ex2, ey2 = extract_coords(r, lc)
        arow = asg_ref[pl.ds(r, 1), :]
        a = jnp.sum(jnp.where(lc, arow, 0.0)).astype(i32)
        gtrow = gtall_ref[pl.ds(a, 1), :]

        def gext(lane):
            return jnp.sum(jnp.where(li == lane, gtrow, 0.0),
                           axis=1, keepdims=True)
        gx1, gy1, gx2, gy2 = gext(24), gext(25), gext(26), gext(27)

        ex_w = ex2 - ex1 + 1.0
        ex_h = ey2 - ey1 + 1.0
        ex_cx = ex1 + 0.5 * ex_w
        ex_cy = ey1 + 0.5 * ex_h
        gt_w = gx2 - gx1 + 1.0
        gt_h = gy2 - gy1 + 1.0
        gt_cx = gx1 + 0.5 * gt_w
        gt_cy = gy1 + 0.5 * gt_h
        dx = (gt_cx - ex_cx) / ex_w
        dy = (gt_cy - ex_cy) / ex_h
        dw = jnp.log(gt_w / ex_w)
        dh = jnp.log(gt_h / ex_h)

        rois_row = jnp.where(li == 0, ex1,
                    jnp.where(li == 1, ey1,
                     jnp.where(li == 2, ex2, ey2)))
        bbox_row = jnp.where(li == 0, dx,
                    jnp.where(li == 1, dy,
                     jnp.where(li == 2, dw, dh)))
        rois_ref[pl.ds(j, 1), :] = rois_row
        labels_ref[pl.ds(j, 1), :] = gtrow
        bbox_ref[pl.ds(j, 1), :] = bbox_row

        idxb = select_lowest_max(bg_ref)
        rb = idxb // _LANES
        cb = idxb % _LANES
        lcb = li == cb
        bx1, by1, bx2, by2 = extract_coords(rb, lcb)
        rois_rowb = jnp.where(li == 0, bx1,
                     jnp.where(li == 1, by1,
                      jnp.where(li == 2, bx2, by2)))
        zeros = jnp.zeros((1, _LANES), f32)
        bg_lab = jnp.where(li == 0, 1.0, 0.0)
        rois_ref[pl.ds(_K + j, 1), :] = rois_rowb
        labels_ref[pl.ds(_K + j, 1), :] = bg_lab
        bbox_ref[pl.ds(_K + j, 1), :] = zeros
        return 0

    lax.fori_loop(0, _K, step, 0)


def kernel(proposals, bounding_boxes, labels):
    f32 = jnp.float32
    p = jnp.concatenate([proposals[0], bounding_boxes[0]], axis=0)
    pp = jnp.pad(p, ((0, _NPAD - _N), (0, 0)))
    planes = pp.T.reshape(4, _ROWS, _LANES)
    gt = bounding_boxes[0]
    lab = labels[0]
    gtall = jnp.zeros((_G, _LANES), f32)
    gtall = gtall.at[:, :_C].set(lab)
    gtall = gtall.at[:, 24:28].set(gt)
    gt_smem = gt.T

    out_shape = [jax.ShapeDtypeStruct((2 * _K, _LANES), f32)] * 3
    rois, labels_out, bbox = pl.pallas_call(
        _proposal_target_kernel,
        out_shape=out_shape,
        in_specs=[
            pl.BlockSpec(memory_space=pltpu.SMEM),
            pl.BlockSpec(memory_space=pltpu.VMEM),
            pl.BlockSpec(memory_space=pltpu.VMEM),
        ],
        out_specs=[pl.BlockSpec(memory_space=pltpu.VMEM)] * 3,
        scratch_shapes=[
            pltpu.VMEM((_ROWS, _LANES), f32),
            pltpu.VMEM((_ROWS, _LANES), f32),
            pltpu.VMEM((_ROWS, _LANES), f32),
        ],
    )(gt_smem, planes, gtall)
    return (rois[None, :, :4], labels_out[None, :, :_C], bbox[None, :, :4])
